# baseline (speedup 1.0000x reference)
import functools

import jax
import jax.numpy as jnp
from jax import lax
from jax.experimental import pallas as pl
from jax.experimental.pallas import tpu as pltpu
from jax.experimental.pallas import tpu_sc as plsc

B = 16384
V = 1000000
D = 128

_info = plsc.get_sparse_core_info()
NC, NS = _info.num_cores, _info.num_subcores
NW = NC * NS
B_PER_W = B // NW


NSUB = 4
SUB = B_PER_W // NSUB


def _make_gather():
    mesh = plsc.VectorSubcoreMesh(core_axis_name="c", subcore_axis_name="s")

    @functools.partial(
        pl.kernel,
        mesh=mesh,
        out_type=jax.ShapeDtypeStruct((B, D), jnp.float32),
        scratch_types=[
            pltpu.VMEM((B_PER_W,), jnp.int32),
            pltpu.VMEM((B_PER_W, D), jnp.float32),
            pltpu.SemaphoreType.DMA,
        ],
    )
    def gather_k(table_hbm, idx_hbm, out_hbm, idx_v, rows_v, sem):
        wid = lax.axis_index("s") * NC + lax.axis_index("c")
        base = wid * B_PER_W
        pltpu.sync_copy(idx_hbm.at[pl.ds(base, B_PER_W)], idx_v)
        pltpu.async_copy(table_hbm.at[idx_v], rows_v, sem).wait()
        pltpu.sync_copy(rows_v, out_hbm.at[pl.ds(base, B_PER_W)])

    return gather_k


_gather = _make_gather()

BK = 4096


def _mlp_body(e_ref, w1_ref, b1_ref, w2_ref, b2_ref, w3_ref, b3_ref, o_ref):
    dn = (((1,), (1,)), ((), ()))
    f32 = jnp.float32
    bf = jnp.bfloat16
    e = e_ref[...].astype(bf)
    h = lax.dot_general(e, w1_ref[...].astype(bf), dn, preferred_element_type=f32)
    h = jnp.maximum(h + b1_ref[...], 0.0).astype(bf)
    h = lax.dot_general(h, w2_ref[...].astype(bf), dn, preferred_element_type=f32)
    h = jnp.maximum(h + b2_ref[...], 0.0).astype(bf)
    h = lax.dot_general(w3_ref[...].astype(bf), h, dn, preferred_element_type=f32)
    o_ref[...] = h + b3_ref[...]


def _mlp_t(e, W1, b1, W2, b2, W3, b3):
    grid = (B // BK,)
    full = lambda shape: pl.BlockSpec(shape, lambda i: (0, 0))
    return pl.pallas_call(
        _mlp_body,
        grid=grid,
        in_specs=[
            pl.BlockSpec((BK, D), lambda i: (i, 0)),
            full((128, D)),
            full((1, 128)),
            full((64, 128)),
            full((1, 64)),
            full((32, 64)),
            full((32, 1)),
        ],
        out_specs=pl.BlockSpec((32, BK), lambda i: (0, i)),
        out_shape=jax.ShapeDtypeStruct((32, B), jnp.float32),
    )(e, W1, b1.reshape(1, 128), W2, b2.reshape(1, 64), W3, b3.reshape(32, 1))


@jax.jit
def kernel(x, emb, W1, b1, W2, b2, W3, b3):
    idx = x.reshape(B)
    e = _gather(emb, idx)
    return _mlp_t(e, W1, b1, W2, b2, W3, b3).T


# speedup vs baseline: 1.0196x; 1.0196x over previous
import functools

import jax
import jax.numpy as jnp
from jax import lax
from jax.experimental import pallas as pl
from jax.experimental.pallas import tpu as pltpu
from jax.experimental.pallas import tpu_sc as plsc

B = 16384
V = 1000000
D = 128

_info = plsc.get_sparse_core_info()
NC, NS = _info.num_cores, _info.num_subcores
NW = NC * NS
B_PER_W = B // NW


def _make_gather():
    mesh = plsc.VectorSubcoreMesh(core_axis_name="c", subcore_axis_name="s")

    @functools.partial(
        pl.kernel,
        mesh=mesh,
        out_type=jax.ShapeDtypeStruct((B, D), jnp.float32),
        scratch_types=[
            pltpu.VMEM((B_PER_W,), jnp.int32),
            pltpu.VMEM((B_PER_W, D), jnp.float32),
            pltpu.SemaphoreType.DMA,
        ],
    )
    def gather_k(table_hbm, idx_hbm, out_hbm, idx_v, rows_v, sem):
        wid = lax.axis_index("s") * NC + lax.axis_index("c")
        base = wid * B_PER_W
        pltpu.sync_copy(idx_hbm.at[pl.ds(base, B_PER_W)], idx_v)
        pltpu.async_copy(table_hbm.at[idx_v], rows_v, sem).wait()
        pltpu.sync_copy(rows_v, out_hbm.at[pl.ds(base, B_PER_W)])

    return gather_k


_gather = _make_gather()

BK = 8192


def _mlp_body(e_ref, w1_ref, b1_ref, w2_ref, b2_ref, w3_ref, b3_ref, o_ref):
    dn = (((1,), (1,)), ((), ()))
    f32 = jnp.float32
    bf = jnp.bfloat16
    e = e_ref[...].astype(bf)
    h = lax.dot_general(e, w1_ref[...].astype(bf), dn, preferred_element_type=f32)
    h = jnp.maximum(h + b1_ref[...], 0.0).astype(bf)
    h = lax.dot_general(h, w2_ref[...].astype(bf), dn, preferred_element_type=f32)
    h = jnp.maximum(h + b2_ref[...], 0.0).astype(bf)
    h = lax.dot_general(w3_ref[...].astype(bf), h, dn, preferred_element_type=f32)
    o_ref[...] = h + b3_ref[...]


def _mlp_t(e, W1, b1, W2, b2, W3, b3):
    grid = (B // BK,)
    full = lambda shape: pl.BlockSpec(shape, lambda i: (0, 0))
    return pl.pallas_call(
        _mlp_body,
        grid=grid,
        in_specs=[
            pl.BlockSpec((BK, D), lambda i: (i, 0)),
            full((128, D)),
            full((1, 128)),
            full((64, 128)),
            full((1, 64)),
            full((32, 64)),
            full((32, 1)),
        ],
        out_specs=pl.BlockSpec((32, BK), lambda i: (0, i)),
        out_shape=jax.ShapeDtypeStruct((32, B), jnp.float32),
    )(e, W1, b1.reshape(1, 128), W2, b2.reshape(1, 64), W3, b3.reshape(32, 1))


@jax.jit
def kernel(x, emb, W1, b1, W2, b2, W3, b3):
    idx = x.reshape(B)
    e = _gather(emb, idx)
    return _mlp_t(e, W1, b1, W2, b2, W3, b3).T
